# Initial kernel scaffold; baseline (speedup 1.0000x reference)
import jax
import jax.numpy as jnp
from jax.experimental import pallas as pl


def kernel(inputs, W):
    raise NotImplementedError("write your pallas kernel here")


# speedup vs baseline: 1.6682x; 1.6682x over previous
import functools

import jax
import jax.numpy as jnp
from jax import lax
from jax.experimental import pallas as pl
from jax.experimental.pallas import tpu as pltpu
from jax.experimental.pallas import tpu_sc as plsc

K = 8192
D = 256
B = 8
L = 1024
T = B * L
KB = 1024
NKB = K // KB

def _dist_body(x_ref, w_ref, idx_ref, minv_ref, bestv, besti):
    kb = pl.program_id(1)
    X = x_ref[0]
    Wb = w_ref[...]
    mm = lax.dot_general(Wb, X, (((1,), (0,)), ((), ())),
                         preferred_element_type=jnp.float32)
    xsq = jnp.sum(X * X, axis=0, keepdims=True)
    wsq = jnp.sum(Wb * Wb, axis=1, keepdims=True)
    d = (xsq + wsq) - 2.0 * mm
    bm = jnp.min(d, axis=0, keepdims=True)
    iota = lax.broadcasted_iota(jnp.int32, d.shape, 0) + kb * KB
    bi = jnp.min(jnp.where(d == bm, iota, K), axis=0, keepdims=True)

    @pl.when(kb == 0)
    def _():
        bestv[...] = bm
        besti[...] = bi

    @pl.when(kb > 0)
    def _():
        upd = bm < bestv[...]
        bestv[...] = jnp.where(upd, bm, bestv[...])
        besti[...] = jnp.where(upd, bi, besti[...])

    @pl.when(kb == NKB - 1)
    def _():
        idx_ref[0] = besti[...]
        minv_ref[0] = bestv[...]


_dist_call = pl.pallas_call(
    _dist_body,
    grid=(B, NKB),
    in_specs=[pl.BlockSpec((1, D, L), lambda b, kb: (b, 0, 0)),
              pl.BlockSpec((KB, D), lambda b, kb: (kb, 0))],
    out_specs=[pl.BlockSpec((1, 1, L), lambda b, kb: (b, 0, 0)),
               pl.BlockSpec((1, 1, L), lambda b, kb: (b, 0, 0))],
    out_shape=[jax.ShapeDtypeStruct((B, 1, L), jnp.int32),
               jax.ShapeDtypeStruct((B, 1, L), jnp.float32)],
    scratch_shapes=[pltpu.VMEM((1, L), jnp.float32),
                    pltpu.VMEM((1, L), jnp.int32)],
    compiler_params=pltpu.CompilerParams(
        dimension_semantics=("arbitrary", "arbitrary")),
)

_NC, _NS = 2, 16
_NW = _NC * _NS
_TPW = T // _NW
_BPW = K // _NS

@functools.cache
def _make_sc_gather_hist():
    mesh = plsc.VectorSubcoreMesh(core_axis_name="c", subcore_axis_name="s")
    return functools.partial(
        pl.kernel,
        out_type=(jax.ShapeDtypeStruct((T, D), jnp.float32),
                  jax.ShapeDtypeStruct((_NC, K), jnp.float32)),
        mesh=mesh,
        scratch_types=[
            pltpu.VMEM((_TPW,), jnp.int32),
            pltpu.VMEM((_TPW, D), jnp.float32),
            pltpu.VMEM((K,), jnp.float32),
            pltpu.VMEM((_BPW,), jnp.float32),
            pltpu.VMEM((_BPW,), jnp.float32),
            pltpu.VMEM_SHARED((_NS, K), jnp.float32),
            pltpu.SemaphoreType.DMA,
        ],
        compiler_params=pltpu.CompilerParams(needs_layout_passes=False),
    )(_sc_gather_hist_body)


def _sc_gather_hist_body(w_hbm, idx_hbm, outg_hbm, counts_hbm,
                         idx_v, rows_v, cnt_v, acc_v, tmp_v, shared, sem):
    c = lax.axis_index("c")
    s = lax.axis_index("s")
    wid = s * _NC + c
    base = wid * _TPW
    pltpu.sync_copy(idx_hbm.at[pl.ds(base, _TPW)], idx_v)
    gather = pltpu.async_copy(w_hbm.at[idx_v], rows_v, sem)

    def _zero(i, _):
        cnt_v[pl.ds(i * 16, 16)] = jnp.zeros((16,), jnp.float32)
        return 0
    lax.fori_loop(0, K // 16, _zero, 0)
    ones = jnp.ones((16,), jnp.float32)

    def _scat(i, _):
        iv = idx_v[pl.ds(i * 16, 16)]
        plsc.addupdate_scatter(cnt_v, [iv], ones)
        return 0
    lax.fori_loop(0, _TPW // 16, _scat, 0)

    pltpu.sync_copy(cnt_v, shared.at[s])
    plsc.subcore_barrier()
    bbase = s * _BPW

    def _zero2(i, _):
        acc_v[pl.ds(i * 16, 16)] = jnp.zeros((16,), jnp.float32)
        return 0
    lax.fori_loop(0, _BPW // 16, _zero2, 0)

    def _red(w, _):
        pltpu.sync_copy(shared.at[w, pl.ds(bbase, _BPW)], tmp_v)
        for j in range(_BPW // 16):
            sl = pl.ds(j * 16, 16)
            acc_v[sl] = acc_v[sl] + tmp_v[sl]
        return 0
    lax.fori_loop(0, _NS, _red, 0)
    pltpu.sync_copy(acc_v, counts_hbm.at[c, pl.ds(bbase, _BPW)])

    gather.wait()
    pltpu.sync_copy(rows_v, outg_hbm.at[pl.ds(base, _TPW)])


def _finish_body(cnt_ref, minv_ref, loss_ref, pplx_ref, dcr_ref):
    c2 = cnt_ref[...]
    c = c2[0] + c2[1]
    p = c * (1.0 / T)
    ent = jnp.sum(p * jnp.log(p + 1e-10))
    pplx = jnp.exp(-ent)
    active = jnp.sum(jnp.where(c > 0, 1.0, 0.0))
    dcr = 1.0 - active * (1.0 / K)
    ssum = jnp.sum(minv_ref[...])
    loss = 1.25 * (ssum * (1.0 / (T * D)))
    loss_ref[...] = jnp.reshape(loss, (1, 1))
    pplx_ref[...] = jnp.reshape(pplx, (1, 1))
    dcr_ref[...] = jnp.reshape(dcr, (1, 1))


_finish_call = pl.pallas_call(
    _finish_body,
    out_shape=[jax.ShapeDtypeStruct((1, 1), jnp.float32)] * 3,
)


def kernel(inputs, W):
    idx8, minv8 = _dist_call(inputs, W)
    idx_flat = idx8.reshape(T)
    outg, counts2 = _make_sc_gather_hist()(W, idx_flat)
    loss, pplx, dcr = _finish_call(counts2.reshape(_NC, 64, 128), minv8)
    quant = outg.reshape(B, L, D).transpose(0, 2, 1)
    return (loss.reshape(()), quant, pplx.reshape(()), dcr.reshape(()))
